# baseline (speedup 1.0000x reference)
import jax
import jax.numpy as jnp
from jax.experimental import pallas as pl

N = 100000


def _gcn_conv(x, src, dst, ew, W, b, num_nodes):
    loop = jnp.arange(num_nodes, dtype=src.dtype)
    s = jnp.concatenate([src, loop])
    d = jnp.concatenate([dst, loop])
    w = jnp.concatenate([ew, jnp.ones((num_nodes,), dtype=ew.dtype)])
    deg = jnp.zeros((num_nodes,), dtype=ew.dtype).at[d].add(w)
    deg_safe = jnp.where(deg > 0, deg, 1.0)
    dinv = jnp.where(deg > 0, deg_safe ** -0.5, 0.0)
    norm = dinv[s] * w * dinv[d]
    xw = x @ W
    msg = xw[s] * norm[:, None]
    out = jnp.zeros((num_nodes, xw.shape[1]), dtype=xw.dtype).at[d].add(msg)
    return out + b


def kernel(x, edge_index, edge_weight, W1, b1, W2, b2, Wl, bl):
    src, dst = edge_index[0], edge_index[1]
    h = jax.nn.relu(_gcn_conv(x, src, dst, edge_weight, W1, b1, N))
    h = jax.nn.relu(_gcn_conv(h, src, dst, edge_weight, W2, b2, N))
    logits = h @ Wl + bl
    return jax.nn.log_softmax(logits, axis=-1)


# speedup vs baseline: 19.6448x; 19.6448x over previous
import functools

import jax
import jax.numpy as jnp
from jax import lax
from jax.experimental import pallas as pl
from jax.experimental.pallas import tpu as pltpu
from jax.experimental.pallas import tpu_sc as plsc

N = 100000
NP = 102400
E = 3200000
EP = 16 * 1568 * 128
ER = EP // 128
TILE_ROWS = 1568
PWR = 8
PB = PWR * 128
PNWIN = TILE_ROWS // PWR
WIN_ROWS = 16

DEG_TILE_ROWS = ER // 32
DEG_NWIN = DEG_TILE_ROWS // WIN_ROWS

TB = 2048

_mesh = plsc.VectorSubcoreMesh(core_axis_name="c", subcore_axis_name="s")


def _deg_body(dstr_hbm, wr2_hbm, degp_hbm, acc, dst_v, w_v, zbuf):
    c = lax.axis_index("c")
    s = lax.axis_index("s")
    wid = c * 16 + s

    @pl.when(s == 0)
    def _():
        @pl.loop(0, 6400 // 16)
        def _(i):
            zbuf[pl.ds(i * 16, 16)] = jnp.zeros((16,), jnp.float32)

        @pl.loop(0, 16)
        def _(b):
            pltpu.sync_copy(zbuf, acc.at[pl.ds(b * 6400, 6400)])

    plsc.subcore_barrier()

    row0 = wid * DEG_TILE_ROWS

    @pl.loop(0, DEG_NWIN)
    def _(i):
        rw = row0 + i * WIN_ROWS
        pltpu.sync_copy(dstr_hbm.at[pl.ds(rw, WIN_ROWS)], dst_v)
        pltpu.sync_copy(wr2_hbm.at[pl.ds(rw, WIN_ROWS)], w_v)
        for r in range(WIN_ROWS):
            pltpu.sync_copy(w_v.at[r], acc.at[dst_v.at[r]], add=True)

    plsc.subcore_barrier()

    @pl.when(s == 0)
    def _():
        pltpu.sync_copy(acc, degp_hbm.at[c])


def _deg_call(dstr, wr2):
    return pl.kernel(
        _deg_body,
        out_type=jax.ShapeDtypeStruct((2, NP), jnp.float32),
        mesh=_mesh,
        scratch_types=[
            pltpu.VMEM_SHARED((NP,), jnp.float32),
            pltpu.VMEM((WIN_ROWS, 128), jnp.int32),
            pltpu.VMEM((WIN_ROWS, 128), jnp.float32),
            pltpu.VMEM((6400,), jnp.float32),
        ],
    )(dstr, wr2)


def _prop_body(K, feat_hbm, srcr_hbm, dstr_hbm, wp_hbm, out_hbm,
               acc, src_v, dst_v, w_v, rows_v, zbuf, sem):
    c = lax.axis_index("c")
    s = lax.axis_index("s")
    row0 = s * TILE_ROWS
    zrow0 = s * (NP // 16)

    @pl.loop(0, 128)
    def _(i):
        zbuf[i, :] = jnp.zeros((16,), jnp.float32)

    for j in range(K // 2):
        base = (c + 2 * j) * NP

        @pl.loop(0, 50)
        def _(b):
            pltpu.sync_copy(zbuf, acc.at[pl.ds(zrow0 + b * 128, 128)])

        plsc.subcore_barrier()

        @pl.loop(0, PNWIN)
        def _(i):
            rw = row0 + i * PWR
            pltpu.sync_copy(srcr_hbm.at[pl.ds(rw, PWR)], src_v)
            pltpu.sync_copy(dstr_hbm.at[pl.ds(rw, PWR)], dst_v)
            pltpu.sync_copy(wp_hbm.at[pl.ds(rw * 128, PB)], w_v)

            base_v = jnp.full((16,), base, jnp.int32)

            @pl.loop(0, 128, step=16)
            def _(q):
                for r in range(PWR):
                    src_v[r, pl.ds(q, 16)] = src_v[r, pl.ds(q, 16)] + base_v

            descs = [
                pltpu.async_copy(
                    feat_hbm.at[src_v.at[r]],
                    rows_v.at[pl.ds(r * 128, 128)],
                    sem,
                )
                for r in range(PWR)
            ]
            for d in descs:
                d.wait()

            @pl.loop(0, PB, step=16)
            def _(q):
                wv = w_v[pl.ds(q, 16)]
                for u in range(16):
                    rows_v[q + u, :] = rows_v[q + u, :] * wv[u]

            for r in range(PWR):
                pltpu.sync_copy(
                    rows_v.at[pl.ds(r * 128, 128)],
                    acc.at[dst_v.at[r]],
                    add=True,
                )

        plsc.subcore_barrier()

        @pl.when(s == 0)
        def _():
            pltpu.sync_copy(acc, out_hbm.at[pl.ds(base, NP)])

        plsc.subcore_barrier()


def _prop_call(K, feat, srcr, dstr, wp):
    return pl.kernel(
        functools.partial(_prop_body, K),
        out_type=jax.ShapeDtypeStruct((K * NP, 16), jnp.float32),
        mesh=_mesh,
        compiler_params=pltpu.CompilerParams(use_tc_tiling_on_sc=False),
        scratch_types=[
            pltpu.VMEM_SHARED((NP, 16), jnp.float32),
            pltpu.VMEM((PWR, 128), jnp.int32),
            pltpu.VMEM((PWR, 128), jnp.int32),
            pltpu.VMEM((PB,), jnp.float32),
            pltpu.VMEM((PB, 16), jnp.float32),
            pltpu.VMEM((128, 16), jnp.float32),
            pltpu.SemaphoreType.DMA,
        ],
    )(feat, srcr, dstr, wp)


def _deg_slice(degp_ref):
    i = pl.program_id(0)
    sl = pl.ds(i * TB, TB)
    return degp_ref[0, sl] + degp_ref[1, sl] + 1.0


def _prep_body(degp_ref, x_ref, featS_ref):
    deg = _deg_slice(degp_ref)
    dv = lax.rsqrt(deg)
    xs = x_ref[...] * dv[:, None]
    featS_ref[0, :, :] = xs[:, :16]
    featS_ref[1, :, :] = jnp.concatenate(
        [xs[:, 16:23], jnp.zeros((TB, 9), jnp.float32)], axis=1)


def _prep_call(degp, x):
    return pl.pallas_call(
        _prep_body,
        grid=(NP // TB,),
        in_specs=[
            pl.BlockSpec((2, NP), lambda i: (0, 0)),
            pl.BlockSpec((TB, 23), lambda i: (i, 0)),
        ],
        out_specs=pl.BlockSpec((2, TB, 16), lambda i: (0, i, 0)),
        out_shape=jax.ShapeDtypeStruct((2, NP, 16), jnp.float32),
    )(degp, x)


def _mid_body(p1_ref, x_ref, degp_ref, W1p_ref, b1_ref, W2_ref,
              tS_ref, self2_ref):
    deg = _deg_slice(degp_ref)
    dv = lax.rsqrt(deg)
    dv2 = 1.0 / deg
    p1 = p1_ref[...]
    xcol = jnp.concatenate([p1[0], p1[1]], axis=1)
    x32 = jnp.concatenate(
        [x_ref[...], jnp.zeros((TB, 9), jnp.float32)], axis=1)
    pre = xcol * dv[:, None] + x32 * dv2[:, None]
    h1 = jnp.maximum(
        jnp.dot(pre, W1p_ref[...], preferred_element_type=jnp.float32)
        + b1_ref[...], 0.0)
    t = jnp.dot(h1, W2_ref[...], preferred_element_type=jnp.float32)
    ts = t * dv[:, None]
    for k in range(4):
        tS_ref[k, :, :] = ts[:, 16 * k:16 * (k + 1)]
    self2_ref[...] = t * dv2[:, None]


def _mid_call(p1, x, degp, W1p, b1r, W2):
    return pl.pallas_call(
        _mid_body,
        grid=(NP // TB,),
        in_specs=[
            pl.BlockSpec((2, TB, 16), lambda i: (0, i, 0)),
            pl.BlockSpec((TB, 23), lambda i: (i, 0)),
            pl.BlockSpec((2, NP), lambda i: (0, 0)),
            pl.BlockSpec((32, 128), lambda i: (0, 0)),
            pl.BlockSpec((1, 128), lambda i: (0, 0)),
            pl.BlockSpec((128, 64), lambda i: (0, 0)),
        ],
        out_specs=[
            pl.BlockSpec((4, TB, 16), lambda i: (0, i, 0)),
            pl.BlockSpec((TB, 64), lambda i: (i, 0)),
        ],
        out_shape=[
            jax.ShapeDtypeStruct((4, NP, 16), jnp.float32),
            jax.ShapeDtypeStruct((NP, 64), jnp.float32),
        ],
    )(p1, x, degp, W1p, b1r, W2)


def _final_body(p2_ref, self2_ref, degp_ref, b2_ref, Wl_ref, bl_ref, out_ref):
    deg = _deg_slice(degp_ref)
    dv = lax.rsqrt(deg)
    p2 = p2_ref[...]
    xcol = jnp.concatenate([p2[0], p2[1], p2[2], p2[3]], axis=1)
    h2 = jnp.maximum(xcol * dv[:, None] + self2_ref[...] + b2_ref[...], 0.0)
    logits = jnp.dot(h2, Wl_ref[...],
                     preferred_element_type=jnp.float32) + bl_ref[...]
    m = jnp.max(logits, axis=1, keepdims=True)
    lse = m + jnp.log(jnp.sum(jnp.exp(logits - m), axis=1, keepdims=True))
    out_ref[...] = logits - lse


def _final_call(p2, self2, degp, b2r, Wl, blr):
    return pl.pallas_call(
        _final_body,
        grid=(NP // TB,),
        in_specs=[
            pl.BlockSpec((4, TB, 16), lambda i: (0, i, 0)),
            pl.BlockSpec((TB, 64), lambda i: (i, 0)),
            pl.BlockSpec((2, NP), lambda i: (0, 0)),
            pl.BlockSpec((1, 64), lambda i: (0, 0)),
            pl.BlockSpec((64, 64), lambda i: (0, 0)),
            pl.BlockSpec((1, 64), lambda i: (0, 0)),
        ],
        out_specs=pl.BlockSpec((TB, 64), lambda i: (i, 0)),
        out_shape=jax.ShapeDtypeStruct((NP, 64), jnp.float32),
    )(p2, self2, degp, b2r, Wl, blr)


def kernel(x, edge_index, edge_weight, W1, b1, W2, b2, Wl, bl):
    src = edge_index[0]
    dst = edge_index[1]
    pad = EP - E
    pad_idx = (jnp.arange(pad, dtype=jnp.int32) % N)
    srcr = jnp.concatenate([src, pad_idx]).reshape(ER, 128)
    dstr = jnp.concatenate([dst, pad_idx]).reshape(ER, 128)
    wp = jnp.concatenate([edge_weight, jnp.zeros((pad,), jnp.float32)])
    wr2 = wp.reshape(ER, 128)

    xp = jnp.concatenate(
        [x, jnp.zeros((NP - N, 23), jnp.float32)], axis=0)
    degp = _deg_call(dstr, wr2)
    featS = _prep_call(degp, xp)
    p1 = _prop_call(2, featS.reshape(2 * NP, 16), srcr, dstr, wp)
    W1p = jnp.concatenate([W1, jnp.zeros((9, 128), jnp.float32)], axis=0)
    tS, self2 = _mid_call(p1.reshape(2, NP, 16), xp, degp, W1p,
                          b1.reshape(1, 128), W2)
    p2 = _prop_call(4, tS.reshape(4 * NP, 16), srcr, dstr, wp)
    outp = _final_call(p2.reshape(4, NP, 16), self2, degp,
                       b2.reshape(1, 64), Wl, bl.reshape(1, 64))
    return outp[:N]


# speedup vs baseline: 28.1861x; 1.4348x over previous
import functools

import jax
import jax.numpy as jnp
from jax import lax
from jax.experimental import pallas as pl
from jax.experimental.pallas import tpu as pltpu
from jax.experimental.pallas import tpu_sc as plsc

N = 100000
NP = 102400
E = 3200000
EP = 16 * 1568 * 128
ER = EP // 128
TILE_ROWS = 1568
PWR = 4
PB = PWR * 128
PNWIN = TILE_ROWS // PWR
WIN_ROWS = 16

DEG_TILE_ROWS = ER // 32
DEG_NWIN = DEG_TILE_ROWS // WIN_ROWS

TB = 2048

_mesh = plsc.VectorSubcoreMesh(core_axis_name="c", subcore_axis_name="s")


def _deg_body(dstr_hbm, wr2_hbm, degp_hbm, acc, dst_v, w_v, zbuf):
    c = lax.axis_index("c")
    s = lax.axis_index("s")
    wid = c * 16 + s

    @pl.when(s == 0)
    def _():
        @pl.loop(0, 6400 // 16)
        def _(i):
            zbuf[pl.ds(i * 16, 16)] = jnp.zeros((16,), jnp.float32)

        @pl.loop(0, 16)
        def _(b):
            pltpu.sync_copy(zbuf, acc.at[pl.ds(b * 6400, 6400)])

    plsc.subcore_barrier()

    row0 = wid * DEG_TILE_ROWS

    @pl.loop(0, DEG_NWIN)
    def _(i):
        rw = row0 + i * WIN_ROWS
        pltpu.sync_copy(dstr_hbm.at[pl.ds(rw, WIN_ROWS)], dst_v)
        pltpu.sync_copy(wr2_hbm.at[pl.ds(rw, WIN_ROWS)], w_v)
        for r in range(WIN_ROWS):
            pltpu.sync_copy(w_v.at[r], acc.at[dst_v.at[r]], add=True)

    plsc.subcore_barrier()

    @pl.when(s == 0)
    def _():
        pltpu.sync_copy(acc, degp_hbm.at[c])


def _deg_call(dstr, wr2):
    return pl.kernel(
        _deg_body,
        out_type=jax.ShapeDtypeStruct((2, NP), jnp.float32),
        mesh=_mesh,
        scratch_types=[
            pltpu.VMEM_SHARED((NP,), jnp.float32),
            pltpu.VMEM((WIN_ROWS, 128), jnp.int32),
            pltpu.VMEM((WIN_ROWS, 128), jnp.float32),
            pltpu.VMEM((6400,), jnp.float32),
        ],
    )(dstr, wr2)


def _win_load(srcr_hbm, dstr_hbm, wp_hbm, src_v, dst_v, w_v, rw, sem):
    pltpu.async_copy(srcr_hbm.at[pl.ds(rw, PWR)], src_v, sem)
    pltpu.async_copy(dstr_hbm.at[pl.ds(rw, PWR)], dst_v, sem)
    pltpu.async_copy(wp_hbm.at[pl.ds(rw * 128, PB)], w_v, sem)


def _win_load_wait(srcr_hbm, dstr_hbm, wp_hbm, src_v, dst_v, w_v, rw, sem):
    pltpu.make_async_copy(srcr_hbm.at[pl.ds(rw, PWR)], src_v, sem).wait()
    pltpu.make_async_copy(dstr_hbm.at[pl.ds(rw, PWR)], dst_v, sem).wait()
    pltpu.make_async_copy(wp_hbm.at[pl.ds(rw * 128, PB)], w_v, sem).wait()


def _win_adjust(src_v, base):
    base_v = jnp.full((16,), base, jnp.int32)

    @pl.loop(0, 128, step=16)
    def _(q):
        for r in range(PWR):
            src_v[r, pl.ds(q, 16)] = src_v[r, pl.ds(q, 16)] + base_v


def _win_gather(feat_hbm, src_v, rows_v, sem):
    for r in range(PWR):
        pltpu.async_copy(
            feat_hbm.at[src_v.at[r]], rows_v.at[pl.ds(r * 128, 128)], sem)


def _win_gather_wait(feat_hbm, src_v, rows_v, sem):
    for r in range(PWR):
        pltpu.make_async_copy(
            feat_hbm.at[src_v.at[r]], rows_v.at[pl.ds(r * 128, 128)],
            sem).wait()


def _win_scale(rows_v, w_v):
    @pl.loop(0, PB, step=16)
    def _(q):
        wv = w_v[pl.ds(q, 16)]
        for u in range(16):
            rows_v[q + u, :] = rows_v[q + u, :] * wv[u]


def _win_scatter(acc, dst_v, rows_v):
    for r in range(PWR):
        pltpu.sync_copy(
            rows_v.at[pl.ds(r * 128, 128)], acc.at[dst_v.at[r]], add=True)


def _prop_body(K, feat_hbm, srcr_hbm, dstr_hbm, wp_hbm, out_hbm,
               acc, src_v, dst_v, w_v, rows_v, zbuf, sem_i, sem_g):
    c = lax.axis_index("c")
    s = lax.axis_index("s")
    row0 = s * TILE_ROWS
    zrow0 = s * (NP // 16)

    @pl.loop(0, 128)
    def _(i):
        zbuf[i, :] = jnp.zeros((16,), jnp.float32)

    def win_rw(i):
        return row0 + i * PWR

    for j in range(K // 2):
        base = (c + 2 * j) * NP

        @pl.loop(0, 50)
        def _(b):
            pltpu.sync_copy(zbuf, acc.at[pl.ds(zrow0 + b * 128, 128)])

        plsc.subcore_barrier()

        _win_load(srcr_hbm, dstr_hbm, wp_hbm,
                  src_v.at[0], dst_v.at[0], w_v.at[0], win_rw(0), sem_i)
        _win_load_wait(srcr_hbm, dstr_hbm, wp_hbm,
                       src_v.at[0], dst_v.at[0], w_v.at[0], win_rw(0), sem_i)
        _win_adjust(src_v.at[0], base)
        _win_gather(feat_hbm, src_v.at[0], rows_v.at[0], sem_g)
        _win_load(srcr_hbm, dstr_hbm, wp_hbm,
                  src_v.at[1], dst_v.at[1], w_v.at[1], win_rw(1), sem_i)

        @pl.loop(0, PNWIN // 2)
        def _(t):
            i0 = 2 * t
            _win_gather_wait(feat_hbm, src_v.at[0], rows_v.at[0], sem_g)
            _win_load_wait(srcr_hbm, dstr_hbm, wp_hbm,
                           src_v.at[1], dst_v.at[1], w_v.at[1],
                           win_rw(i0 + 1), sem_i)
            _win_adjust(src_v.at[1], base)
            _win_gather(feat_hbm, src_v.at[1], rows_v.at[1], sem_g)
            _win_scale(rows_v.at[0], w_v.at[0])
            _win_scatter(acc, dst_v.at[0], rows_v.at[0])

            @pl.when(t < PNWIN // 2 - 1)
            def _():
                _win_load(srcr_hbm, dstr_hbm, wp_hbm,
                          src_v.at[0], dst_v.at[0], w_v.at[0],
                          win_rw(i0 + 2), sem_i)

            _win_gather_wait(feat_hbm, src_v.at[1], rows_v.at[1], sem_g)

            @pl.when(t < PNWIN // 2 - 1)
            def _():
                _win_load_wait(srcr_hbm, dstr_hbm, wp_hbm,
                               src_v.at[0], dst_v.at[0], w_v.at[0],
                               win_rw(i0 + 2), sem_i)
                _win_adjust(src_v.at[0], base)
                _win_gather(feat_hbm, src_v.at[0], rows_v.at[0], sem_g)

            _win_scale(rows_v.at[1], w_v.at[1])
            _win_scatter(acc, dst_v.at[1], rows_v.at[1])

            @pl.when(t < PNWIN // 2 - 1)
            def _():
                _win_load(srcr_hbm, dstr_hbm, wp_hbm,
                          src_v.at[1], dst_v.at[1], w_v.at[1],
                          win_rw(i0 + 3), sem_i)

        plsc.subcore_barrier()

        @pl.when(s == 0)
        def _():
            pltpu.sync_copy(acc, out_hbm.at[pl.ds(base, NP)])

        plsc.subcore_barrier()


def _prop_call(K, feat, srcr, dstr, wp):
    return pl.kernel(
        functools.partial(_prop_body, K),
        out_type=jax.ShapeDtypeStruct((K * NP, 16), jnp.float32),
        mesh=_mesh,
        compiler_params=pltpu.CompilerParams(use_tc_tiling_on_sc=False),
        scratch_types=[
            pltpu.VMEM_SHARED((NP, 16), jnp.float32),
            pltpu.VMEM((2, PWR, 128), jnp.int32),
            pltpu.VMEM((2, PWR, 128), jnp.int32),
            pltpu.VMEM((2, PB), jnp.float32),
            pltpu.VMEM((2, PB, 16), jnp.float32),
            pltpu.VMEM((128, 16), jnp.float32),
            pltpu.SemaphoreType.DMA,
            pltpu.SemaphoreType.DMA,
        ],
    )(feat, srcr, dstr, wp)


def _deg_slice(degp_ref):
    i = pl.program_id(0)
    sl = pl.ds(i * TB, TB)
    return degp_ref[0, sl] + degp_ref[1, sl] + 1.0


def _prep_body(degp_ref, x_ref, featS_ref):
    deg = _deg_slice(degp_ref)
    dv = lax.rsqrt(deg)
    xs = x_ref[...] * dv[:, None]
    featS_ref[0, :, :] = xs[:, :16]
    featS_ref[1, :, :] = jnp.concatenate(
        [xs[:, 16:23], jnp.zeros((TB, 9), jnp.float32)], axis=1)


def _prep_call(degp, x):
    return pl.pallas_call(
        _prep_body,
        grid=(NP // TB,),
        in_specs=[
            pl.BlockSpec((2, NP), lambda i: (0, 0)),
            pl.BlockSpec((TB, 23), lambda i: (i, 0)),
        ],
        out_specs=pl.BlockSpec((2, TB, 16), lambda i: (0, i, 0)),
        out_shape=jax.ShapeDtypeStruct((2, NP, 16), jnp.float32),
    )(degp, x)


def _mid_body(p1_ref, x_ref, degp_ref, W1p_ref, b1_ref, W2_ref,
              tS_ref, self2_ref):
    deg = _deg_slice(degp_ref)
    dv = lax.rsqrt(deg)
    dv2 = 1.0 / deg
    p1 = p1_ref[...]
    xcol = jnp.concatenate([p1[0], p1[1]], axis=1)
    x32 = jnp.concatenate(
        [x_ref[...], jnp.zeros((TB, 9), jnp.float32)], axis=1)
    pre = xcol * dv[:, None] + x32 * dv2[:, None]
    h1 = jnp.maximum(
        jnp.dot(pre, W1p_ref[...], preferred_element_type=jnp.float32)
        + b1_ref[...], 0.0)
    t = jnp.dot(h1, W2_ref[...], preferred_element_type=jnp.float32)
    ts = t * dv[:, None]
    for k in range(4):
        tS_ref[k, :, :] = ts[:, 16 * k:16 * (k + 1)]
    self2_ref[...] = t * dv2[:, None]


def _mid_call(p1, x, degp, W1p, b1r, W2):
    return pl.pallas_call(
        _mid_body,
        grid=(NP // TB,),
        in_specs=[
            pl.BlockSpec((2, TB, 16), lambda i: (0, i, 0)),
            pl.BlockSpec((TB, 23), lambda i: (i, 0)),
            pl.BlockSpec((2, NP), lambda i: (0, 0)),
            pl.BlockSpec((32, 128), lambda i: (0, 0)),
            pl.BlockSpec((1, 128), lambda i: (0, 0)),
            pl.BlockSpec((128, 64), lambda i: (0, 0)),
        ],
        out_specs=[
            pl.BlockSpec((4, TB, 16), lambda i: (0, i, 0)),
            pl.BlockSpec((TB, 64), lambda i: (i, 0)),
        ],
        out_shape=[
            jax.ShapeDtypeStruct((4, NP, 16), jnp.float32),
            jax.ShapeDtypeStruct((NP, 64), jnp.float32),
        ],
    )(p1, x, degp, W1p, b1r, W2)


def _final_body(p2_ref, self2_ref, degp_ref, b2_ref, Wl_ref, bl_ref, out_ref):
    deg = _deg_slice(degp_ref)
    dv = lax.rsqrt(deg)
    p2 = p2_ref[...]
    xcol = jnp.concatenate([p2[0], p2[1], p2[2], p2[3]], axis=1)
    h2 = jnp.maximum(xcol * dv[:, None] + self2_ref[...] + b2_ref[...], 0.0)
    logits = jnp.dot(h2, Wl_ref[...],
                     preferred_element_type=jnp.float32) + bl_ref[...]
    m = jnp.max(logits, axis=1, keepdims=True)
    lse = m + jnp.log(jnp.sum(jnp.exp(logits - m), axis=1, keepdims=True))
    out_ref[...] = logits - lse


def _final_call(p2, self2, degp, b2r, Wl, blr):
    return pl.pallas_call(
        _final_body,
        grid=(NP // TB,),
        in_specs=[
            pl.BlockSpec((4, TB, 16), lambda i: (0, i, 0)),
            pl.BlockSpec((TB, 64), lambda i: (i, 0)),
            pl.BlockSpec((2, NP), lambda i: (0, 0)),
            pl.BlockSpec((1, 64), lambda i: (0, 0)),
            pl.BlockSpec((64, 64), lambda i: (0, 0)),
            pl.BlockSpec((1, 64), lambda i: (0, 0)),
        ],
        out_specs=pl.BlockSpec((TB, 64), lambda i: (i, 0)),
        out_shape=jax.ShapeDtypeStruct((NP, 64), jnp.float32),
    )(p2, self2, degp, b2r, Wl, blr)


def kernel(x, edge_index, edge_weight, W1, b1, W2, b2, Wl, bl):
    src = edge_index[0]
    dst = edge_index[1]
    pad = EP - E
    pad_idx = (jnp.arange(pad, dtype=jnp.int32) % N)
    srcr = jnp.concatenate([src, pad_idx]).reshape(ER, 128)
    dstr = jnp.concatenate([dst, pad_idx]).reshape(ER, 128)
    wp = jnp.concatenate([edge_weight, jnp.zeros((pad,), jnp.float32)])
    wr2 = wp.reshape(ER, 128)

    xp = jnp.concatenate(
        [x, jnp.zeros((NP - N, 23), jnp.float32)], axis=0)
    degp = _deg_call(dstr, wr2)
    featS = _prep_call(degp, xp)
    p1 = _prop_call(2, featS.reshape(2 * NP, 16), srcr, dstr, wp)
    W1p = jnp.concatenate([W1, jnp.zeros((9, 128), jnp.float32)], axis=0)
    tS, self2 = _mid_call(p1.reshape(2, NP, 16), xp, degp, W1p,
                          b1.reshape(1, 128), W2)
    p2 = _prop_call(4, tS.reshape(4 * NP, 16), srcr, dstr, wp)
    outp = _final_call(p2.reshape(4, NP, 16), self2, degp,
                       b2.reshape(1, 64), Wl, bl.reshape(1, 64))
    return outp[:N]


# speedup vs baseline: 32.9429x; 1.1688x over previous
import functools

import jax
import jax.numpy as jnp
from jax import lax
from jax.experimental import pallas as pl
from jax.experimental.pallas import tpu as pltpu
from jax.experimental.pallas import tpu_sc as plsc

N = 100000
NP = 102400
E = 3200000
EP = 16 * 1568 * 128
ER = EP // 128
TILE_ROWS = 1568
PWR = 4
PB = PWR * 128
PNWIN = TILE_ROWS // PWR
WIN_ROWS = 16

DEG_TILE_ROWS = ER // 32
DEG_NWIN = DEG_TILE_ROWS // WIN_ROWS

TB = 2048

_mesh = plsc.VectorSubcoreMesh(core_axis_name="c", subcore_axis_name="s")


def _deg_body(dstr_hbm, wr2_hbm, degp_hbm, acc, dst_v, w_v, zbuf):
    c = lax.axis_index("c")
    s = lax.axis_index("s")
    wid = c * 16 + s

    @pl.when(s == 0)
    def _():
        @pl.loop(0, 6400 // 16)
        def _(i):
            zbuf[pl.ds(i * 16, 16)] = jnp.zeros((16,), jnp.float32)

        @pl.loop(0, 16)
        def _(b):
            pltpu.sync_copy(zbuf, acc.at[pl.ds(b * 6400, 6400)])

    plsc.subcore_barrier()

    row0 = wid * DEG_TILE_ROWS

    @pl.loop(0, DEG_NWIN)
    def _(i):
        rw = row0 + i * WIN_ROWS
        pltpu.sync_copy(dstr_hbm.at[pl.ds(rw, WIN_ROWS)], dst_v)
        pltpu.sync_copy(wr2_hbm.at[pl.ds(rw, WIN_ROWS)], w_v)
        for r in range(WIN_ROWS):
            pltpu.sync_copy(w_v.at[r], acc.at[dst_v.at[r]], add=True)

    plsc.subcore_barrier()

    @pl.when(s == 0)
    def _():
        pltpu.sync_copy(acc, degp_hbm.at[c])


def _deg_call(dstr, wr2):
    return pl.kernel(
        _deg_body,
        out_type=jax.ShapeDtypeStruct((2, NP), jnp.float32),
        mesh=_mesh,
        scratch_types=[
            pltpu.VMEM_SHARED((NP,), jnp.float32),
            pltpu.VMEM((WIN_ROWS, 128), jnp.int32),
            pltpu.VMEM((WIN_ROWS, 128), jnp.float32),
            pltpu.VMEM((6400,), jnp.float32),
        ],
    )(dstr, wr2)


def _prop_body(K, feat_hbm, srcr_hbm, dstr_hbm, wp_hbm, out_hbm,
               acc, src_v, dst_v, w_v, rows_v, zbuf,
               sem_i, sem_d, sem_g, sem_s):
    c = lax.axis_index("c")
    s = lax.axis_index("s")
    row0 = s * TILE_ROWS
    zrow0 = s * (NP // 16)

    @pl.loop(0, 128)
    def _(i):
        zbuf[i, :] = jnp.zeros((16,), jnp.float32)

    def load_srcw(pp, i, fire):
        rw = row0 + i * PWR
        a = pltpu.make_async_copy(
            srcr_hbm.at[pl.ds(rw, PWR)], src_v.at[pp], sem_i)
        b = pltpu.make_async_copy(
            wp_hbm.at[pl.ds(rw * 128, PB)], w_v.at[pp], sem_i)
        if fire:
            a.start(); b.start()
        else:
            a.wait(); b.wait()

    def load_dst(pp, i, fire):
        rw = row0 + i * PWR
        a = pltpu.make_async_copy(
            dstr_hbm.at[pl.ds(rw, PWR)], dst_v.at[pp], sem_d)
        a.start() if fire else a.wait()

    def gather(pp, fire):
        for r in range(PWR):
            a = pltpu.make_async_copy(
                feat_hbm.at[src_v.at[pp].at[r]],
                rows_v.at[pp].at[pl.ds(r * 128, 128)], sem_g)
            a.start() if fire else a.wait()

    def scatter(pp, fire):
        for r in range(PWR):
            if fire:
                pltpu.async_copy(
                    rows_v.at[pp].at[pl.ds(r * 128, 128)],
                    acc.at[dst_v.at[pp].at[r]], sem_s, add=True)
            else:
                pltpu.make_async_copy(
                    rows_v.at[pp].at[pl.ds(r * 128, 128)],
                    acc.at[dst_v.at[pp].at[r]], sem_s).wait()

    def adjust(pp, base):
        base_v = jnp.full((16,), base, jnp.int32)

        @pl.loop(0, 128, step=16)
        def _(q):
            for r in range(PWR):
                src_v[pp, r, pl.ds(q, 16)] = (
                    src_v[pp, r, pl.ds(q, 16)] + base_v)

    def scale(pp):
        @pl.loop(0, PB, step=16)
        def _(q):
            wv = w_v[pp, pl.ds(q, 16)]
            for u in range(16):
                rows_v[pp, q + u, :] = rows_v[pp, q + u, :] * wv[u]

    NW = PNWIN
    for j in range(K // 2):
        base = (c + 2 * j) * NP

        @pl.loop(0, 50)
        def _(b):
            pltpu.sync_copy(zbuf, acc.at[pl.ds(zrow0 + b * 128, 128)])

        plsc.subcore_barrier()

        load_srcw(0, 0, True)
        load_srcw(0, 0, False)
        adjust(0, base)
        gather(0, True)
        load_srcw(1, 1, True)
        load_dst(0, 0, True)

        @pl.loop(0, NW // 2)
        def _(t):
            i0 = 2 * t
            gather(0, False)
            load_srcw(1, i0 + 1, False)
            adjust(1, base)

            @pl.when(t > 0)
            def _():
                scatter(1, False)

            load_dst(1, i0 + 1, True)
            gather(1, True)
            scale(0)
            load_dst(0, i0, False)
            scatter(0, True)

            @pl.when(t < NW // 2 - 1)
            def _():
                load_srcw(0, i0 + 2, True)

            gather(1, False)

            @pl.when(t < NW // 2 - 1)
            def _():
                load_srcw(0, i0 + 2, False)
                adjust(0, base)
                scatter(0, False)
                load_dst(0, i0 + 2, True)
                gather(0, True)

            @pl.when(t == NW // 2 - 1)
            def _():
                scatter(0, False)

            scale(1)
            load_dst(1, i0 + 1, False)
            scatter(1, True)

            @pl.when(t < NW // 2 - 1)
            def _():
                load_srcw(1, i0 + 3, True)

        scatter(1, False)

        plsc.subcore_barrier()

        @pl.when(s == 0)
        def _():
            pltpu.sync_copy(acc, out_hbm.at[pl.ds(base, NP)])

        plsc.subcore_barrier()


def _prop_call(K, feat, srcr, dstr, wp):
    return pl.kernel(
        functools.partial(_prop_body, K),
        out_type=jax.ShapeDtypeStruct((K * NP, 16), jnp.float32),
        mesh=_mesh,
        compiler_params=pltpu.CompilerParams(use_tc_tiling_on_sc=False),
        scratch_types=[
            pltpu.VMEM_SHARED((NP, 16), jnp.float32),
            pltpu.VMEM((2, PWR, 128), jnp.int32),
            pltpu.VMEM((2, PWR, 128), jnp.int32),
            pltpu.VMEM((2, PB), jnp.float32),
            pltpu.VMEM((2, PB, 16), jnp.float32),
            pltpu.VMEM((128, 16), jnp.float32),
            pltpu.SemaphoreType.DMA,
            pltpu.SemaphoreType.DMA,
            pltpu.SemaphoreType.DMA,
            pltpu.SemaphoreType.DMA,
        ],
    )(feat, srcr, dstr, wp)


def _deg_slice(degp_ref):
    i = pl.program_id(0)
    sl = pl.ds(i * TB, TB)
    return degp_ref[0, sl] + degp_ref[1, sl] + 1.0


def _prep_body(degp_ref, x_ref, featS_ref):
    deg = _deg_slice(degp_ref)
    dv = lax.rsqrt(deg)
    xs = x_ref[...] * dv[:, None]
    featS_ref[0, :, :] = xs[:, :16]
    featS_ref[1, :, :] = jnp.concatenate(
        [xs[:, 16:23], jnp.zeros((TB, 9), jnp.float32)], axis=1)


def _prep_call(degp, x):
    return pl.pallas_call(
        _prep_body,
        grid=(NP // TB,),
        in_specs=[
            pl.BlockSpec((2, NP), lambda i: (0, 0)),
            pl.BlockSpec((TB, 23), lambda i: (i, 0)),
        ],
        out_specs=pl.BlockSpec((2, TB, 16), lambda i: (0, i, 0)),
        out_shape=jax.ShapeDtypeStruct((2, NP, 16), jnp.float32),
    )(degp, x)


def _mid_body(p1_ref, x_ref, degp_ref, W1p_ref, b1_ref, W2_ref,
              tS_ref, self2_ref):
    deg = _deg_slice(degp_ref)
    dv = lax.rsqrt(deg)
    dv2 = 1.0 / deg
    p1 = p1_ref[...]
    xcol = jnp.concatenate([p1[0], p1[1]], axis=1)
    x32 = jnp.concatenate(
        [x_ref[...], jnp.zeros((TB, 9), jnp.float32)], axis=1)
    pre = xcol * dv[:, None] + x32 * dv2[:, None]
    h1 = jnp.maximum(
        jnp.dot(pre, W1p_ref[...], preferred_element_type=jnp.float32)
        + b1_ref[...], 0.0)
    t = jnp.dot(h1, W2_ref[...], preferred_element_type=jnp.float32)
    ts = t * dv[:, None]
    for k in range(4):
        tS_ref[k, :, :] = ts[:, 16 * k:16 * (k + 1)]
    self2_ref[...] = t * dv2[:, None]


def _mid_call(p1, x, degp, W1p, b1r, W2):
    return pl.pallas_call(
        _mid_body,
        grid=(NP // TB,),
        in_specs=[
            pl.BlockSpec((2, TB, 16), lambda i: (0, i, 0)),
            pl.BlockSpec((TB, 23), lambda i: (i, 0)),
            pl.BlockSpec((2, NP), lambda i: (0, 0)),
            pl.BlockSpec((32, 128), lambda i: (0, 0)),
            pl.BlockSpec((1, 128), lambda i: (0, 0)),
            pl.BlockSpec((128, 64), lambda i: (0, 0)),
        ],
        out_specs=[
            pl.BlockSpec((4, TB, 16), lambda i: (0, i, 0)),
            pl.BlockSpec((TB, 64), lambda i: (i, 0)),
        ],
        out_shape=[
            jax.ShapeDtypeStruct((4, NP, 16), jnp.float32),
            jax.ShapeDtypeStruct((NP, 64), jnp.float32),
        ],
    )(p1, x, degp, W1p, b1r, W2)


def _final_body(p2_ref, self2_ref, degp_ref, b2_ref, Wl_ref, bl_ref, out_ref):
    deg = _deg_slice(degp_ref)
    dv = lax.rsqrt(deg)
    p2 = p2_ref[...]
    xcol = jnp.concatenate([p2[0], p2[1], p2[2], p2[3]], axis=1)
    h2 = jnp.maximum(xcol * dv[:, None] + self2_ref[...] + b2_ref[...], 0.0)
    logits = jnp.dot(h2, Wl_ref[...],
                     preferred_element_type=jnp.float32) + bl_ref[...]
    m = jnp.max(logits, axis=1, keepdims=True)
    lse = m + jnp.log(jnp.sum(jnp.exp(logits - m), axis=1, keepdims=True))
    out_ref[...] = logits - lse


def _final_call(p2, self2, degp, b2r, Wl, blr):
    return pl.pallas_call(
        _final_body,
        grid=(NP // TB,),
        in_specs=[
            pl.BlockSpec((4, TB, 16), lambda i: (0, i, 0)),
            pl.BlockSpec((TB, 64), lambda i: (i, 0)),
            pl.BlockSpec((2, NP), lambda i: (0, 0)),
            pl.BlockSpec((1, 64), lambda i: (0, 0)),
            pl.BlockSpec((64, 64), lambda i: (0, 0)),
            pl.BlockSpec((1, 64), lambda i: (0, 0)),
        ],
        out_specs=pl.BlockSpec((TB, 64), lambda i: (i, 0)),
        out_shape=jax.ShapeDtypeStruct((NP, 64), jnp.float32),
    )(p2, self2, degp, b2r, Wl, blr)


def kernel(x, edge_index, edge_weight, W1, b1, W2, b2, Wl, bl):
    src = edge_index[0]
    dst = edge_index[1]
    pad = EP - E
    pad_idx = (jnp.arange(pad, dtype=jnp.int32) % N)
    srcr = jnp.concatenate([src, pad_idx]).reshape(ER, 128)
    dstr = jnp.concatenate([dst, pad_idx]).reshape(ER, 128)
    wp = jnp.concatenate([edge_weight, jnp.zeros((pad,), jnp.float32)])
    wr2 = wp.reshape(ER, 128)

    xp = jnp.concatenate(
        [x, jnp.zeros((NP - N, 23), jnp.float32)], axis=0)
    degp = _deg_call(dstr, wr2)
    featS = _prep_call(degp, xp)
    p1 = _prop_call(2, featS.reshape(2 * NP, 16), srcr, dstr, wp)
    W1p = jnp.concatenate([W1, jnp.zeros((9, 128), jnp.float32)], axis=0)
    tS, self2 = _mid_call(p1.reshape(2, NP, 16), xp, degp, W1p,
                          b1.reshape(1, 128), W2)
    p2 = _prop_call(4, tS.reshape(4 * NP, 16), srcr, dstr, wp)
    outp = _final_call(p2.reshape(4, NP, 16), self2, degp,
                       b2.reshape(1, 64), Wl, bl.reshape(1, 64))
    return outp[:N]


# speedup vs baseline: 33.5063x; 1.0171x over previous
import functools

import jax
import jax.numpy as jnp
from jax import lax
from jax.experimental import pallas as pl
from jax.experimental.pallas import tpu as pltpu
from jax.experimental.pallas import tpu_sc as plsc

N = 100000
NP = 102400
E = 3200000
EP = 16 * 1568 * 128
ER = EP // 128
TILE_ROWS = 1568
PWR = 4
PB = PWR * 128
PNWIN = TILE_ROWS // PWR
WIN_ROWS = 16

DEG_TILE_ROWS = ER // 32
DEG_NWIN = DEG_TILE_ROWS // WIN_ROWS

TB = 2048

_mesh = plsc.VectorSubcoreMesh(core_axis_name="c", subcore_axis_name="s")


def _deg_body(dstr_hbm, wr2_hbm, degp_hbm, acc, dst_v, w_v, zbuf):
    c = lax.axis_index("c")
    s = lax.axis_index("s")
    wid = c * 16 + s

    @pl.when(s == 0)
    def _():
        @pl.loop(0, 6400 // 16)
        def _(i):
            zbuf[pl.ds(i * 16, 16)] = jnp.zeros((16,), jnp.float32)

        @pl.loop(0, 16)
        def _(b):
            pltpu.sync_copy(zbuf, acc.at[pl.ds(b * 6400, 6400)])

    plsc.subcore_barrier()

    row0 = wid * DEG_TILE_ROWS

    @pl.loop(0, DEG_NWIN)
    def _(i):
        rw = row0 + i * WIN_ROWS
        pltpu.sync_copy(dstr_hbm.at[pl.ds(rw, WIN_ROWS)], dst_v)
        pltpu.sync_copy(wr2_hbm.at[pl.ds(rw, WIN_ROWS)], w_v)
        for r in range(WIN_ROWS):
            pltpu.sync_copy(w_v.at[r], acc.at[dst_v.at[r]], add=True)

    plsc.subcore_barrier()

    @pl.when(s == 0)
    def _():
        pltpu.sync_copy(acc, degp_hbm.at[c])


def _deg_call(dstr, wr2):
    return pl.kernel(
        _deg_body,
        out_type=jax.ShapeDtypeStruct((2, NP), jnp.float32),
        mesh=_mesh,
        scratch_types=[
            pltpu.VMEM_SHARED((NP,), jnp.float32),
            pltpu.VMEM((WIN_ROWS, 128), jnp.int32),
            pltpu.VMEM((WIN_ROWS, 128), jnp.float32),
            pltpu.VMEM((6400,), jnp.float32),
        ],
    )(dstr, wr2)


def _prop_body(K, feat_hbm, srcr_hbm, dstr_hbm, wp_hbm, out_hbm,
               acc, src_v, dst_v, w_v, rows_v, zbuf,
               sem_i, sem_d, sem_g, sem_s):
    c = lax.axis_index("c")
    s = lax.axis_index("s")
    row0 = s * TILE_ROWS
    zrow0 = s * (NP // 16)

    @pl.loop(0, 128)
    def _(i):
        zbuf[i, :] = jnp.zeros((16,), jnp.float32)

    def load_srcw(pp, i, fire):
        rw = row0 + i * PWR
        a = pltpu.make_async_copy(
            srcr_hbm.at[pl.ds(rw, PWR)], src_v.at[pp], sem_i)
        b = pltpu.make_async_copy(
            wp_hbm.at[pl.ds(rw * 128, PB)], w_v.at[pp], sem_i)
        if fire:
            a.start(); b.start()
        else:
            a.wait(); b.wait()

    def load_dst(pp, i, fire):
        rw = row0 + i * PWR
        a = pltpu.make_async_copy(
            dstr_hbm.at[pl.ds(rw, PWR)], dst_v.at[pp], sem_d)
        a.start() if fire else a.wait()

    def gather(pp, kb, fire):
        for r in range(PWR):
            a = pltpu.make_async_copy(
                feat_hbm.at[kb].at[src_v.at[pp].at[r]],
                rows_v.at[pp].at[pl.ds(r * 128, 128)], sem_g)
            a.start() if fire else a.wait()

    def scatter(pp, fire):
        for r in range(PWR):
            if fire:
                pltpu.async_copy(
                    rows_v.at[pp].at[pl.ds(r * 128, 128)],
                    acc.at[dst_v.at[pp].at[r]], sem_s, add=True)
            else:
                pltpu.make_async_copy(
                    rows_v.at[pp].at[pl.ds(r * 128, 128)],
                    acc.at[dst_v.at[pp].at[r]], sem_s).wait()

    def scale(pp):
        @pl.loop(0, PB, step=16)
        def _(q):
            wv = w_v[pp, pl.ds(q, 16)]
            for u in range(16):
                rows_v[pp, q + u, :] = rows_v[pp, q + u, :] * wv[u]

    NW = PNWIN
    for j in range(K // 2):
        kb = c + 2 * j

        @pl.loop(0, 50)
        def _(b):
            pltpu.sync_copy(zbuf, acc.at[pl.ds(zrow0 + b * 128, 128)])

        plsc.subcore_barrier()

        load_srcw(0, 0, True)
        load_srcw(0, 0, False)
        gather(0, kb, True)
        load_srcw(1, 1, True)
        load_dst(0, 0, True)

        @pl.loop(0, NW // 2)
        def _(t):
            i0 = 2 * t
            gather(0, kb, False)
            load_srcw(1, i0 + 1, False)

            @pl.when(t > 0)
            def _():
                scatter(1, False)

            load_dst(1, i0 + 1, True)
            gather(1, kb, True)
            scale(0)
            load_dst(0, i0, False)
            scatter(0, True)

            @pl.when(t < NW // 2 - 1)
            def _():
                load_srcw(0, i0 + 2, True)

            gather(1, kb, False)

            @pl.when(t < NW // 2 - 1)
            def _():
                load_srcw(0, i0 + 2, False)
                scatter(0, False)
                load_dst(0, i0 + 2, True)
                gather(0, kb, True)

            @pl.when(t == NW // 2 - 1)
            def _():
                scatter(0, False)

            scale(1)
            load_dst(1, i0 + 1, False)
            scatter(1, True)

            @pl.when(t < NW // 2 - 1)
            def _():
                load_srcw(1, i0 + 3, True)

        scatter(1, False)

        plsc.subcore_barrier()

        @pl.when(s == 0)
        def _():
            pltpu.sync_copy(acc, out_hbm.at[kb])

        plsc.subcore_barrier()


def _prop_call(K, feat, srcr, dstr, wp):
    return pl.kernel(
        functools.partial(_prop_body, K),
        out_type=jax.ShapeDtypeStruct((K, NP, 16), jnp.float32),
        mesh=_mesh,
        compiler_params=pltpu.CompilerParams(use_tc_tiling_on_sc=False),
        scratch_types=[
            pltpu.VMEM_SHARED((NP, 16), jnp.float32),
            pltpu.VMEM((2, PWR, 128), jnp.int32),
            pltpu.VMEM((2, PWR, 128), jnp.int32),
            pltpu.VMEM((2, PB), jnp.float32),
            pltpu.VMEM((2, PB, 16), jnp.float32),
            pltpu.VMEM((128, 16), jnp.float32),
            pltpu.SemaphoreType.DMA,
            pltpu.SemaphoreType.DMA,
            pltpu.SemaphoreType.DMA,
            pltpu.SemaphoreType.DMA,
        ],
    )(feat, srcr, dstr, wp)


def _deg_slice(degp_ref):
    i = pl.program_id(0)
    sl = pl.ds(i * TB, TB)
    return degp_ref[0, sl] + degp_ref[1, sl] + 1.0


def _prep_body(degp_ref, x_ref, featS_ref):
    deg = _deg_slice(degp_ref)
    dv = lax.rsqrt(deg)
    xs = x_ref[...] * dv[:, None]
    featS_ref[0, :, :] = xs[:, :16]
    featS_ref[1, :, :] = jnp.concatenate(
        [xs[:, 16:23], jnp.zeros((TB, 9), jnp.float32)], axis=1)


def _prep_call(degp, x):
    return pl.pallas_call(
        _prep_body,
        grid=(NP // TB,),
        in_specs=[
            pl.BlockSpec((2, NP), lambda i: (0, 0)),
            pl.BlockSpec((TB, 23), lambda i: (i, 0)),
        ],
        out_specs=pl.BlockSpec((2, TB, 16), lambda i: (0, i, 0)),
        out_shape=jax.ShapeDtypeStruct((2, NP, 16), jnp.float32),
    )(degp, x)


def _mid_body(p1_ref, x_ref, degp_ref, W1p_ref, b1_ref, W2_ref,
              tS_ref, self2_ref):
    deg = _deg_slice(degp_ref)
    dv = lax.rsqrt(deg)
    dv2 = 1.0 / deg
    p1 = p1_ref[...]
    xcol = jnp.concatenate([p1[0], p1[1]], axis=1)
    x32 = jnp.concatenate(
        [x_ref[...], jnp.zeros((TB, 9), jnp.float32)], axis=1)
    pre = xcol * dv[:, None] + x32 * dv2[:, None]
    h1 = jnp.maximum(
        jnp.dot(pre, W1p_ref[...], preferred_element_type=jnp.float32)
        + b1_ref[...], 0.0)
    t = jnp.dot(h1, W2_ref[...], preferred_element_type=jnp.float32)
    ts = t * dv[:, None]
    for k in range(4):
        tS_ref[k, :, :] = ts[:, 16 * k:16 * (k + 1)]
    self2_ref[...] = t * dv2[:, None]


def _mid_call(p1, x, degp, W1p, b1r, W2):
    return pl.pallas_call(
        _mid_body,
        grid=(NP // TB,),
        in_specs=[
            pl.BlockSpec((2, TB, 16), lambda i: (0, i, 0)),
            pl.BlockSpec((TB, 23), lambda i: (i, 0)),
            pl.BlockSpec((2, NP), lambda i: (0, 0)),
            pl.BlockSpec((32, 128), lambda i: (0, 0)),
            pl.BlockSpec((1, 128), lambda i: (0, 0)),
            pl.BlockSpec((128, 64), lambda i: (0, 0)),
        ],
        out_specs=[
            pl.BlockSpec((4, TB, 16), lambda i: (0, i, 0)),
            pl.BlockSpec((TB, 64), lambda i: (i, 0)),
        ],
        out_shape=[
            jax.ShapeDtypeStruct((4, NP, 16), jnp.float32),
            jax.ShapeDtypeStruct((NP, 64), jnp.float32),
        ],
    )(p1, x, degp, W1p, b1r, W2)


def _final_body(p2_ref, self2_ref, degp_ref, b2_ref, Wl_ref, bl_ref, out_ref):
    deg = _deg_slice(degp_ref)
    dv = lax.rsqrt(deg)
    p2 = p2_ref[...]
    xcol = jnp.concatenate([p2[0], p2[1], p2[2], p2[3]], axis=1)
    h2 = jnp.maximum(xcol * dv[:, None] + self2_ref[...] + b2_ref[...], 0.0)
    logits = jnp.dot(h2, Wl_ref[...],
                     preferred_element_type=jnp.float32) + bl_ref[...]
    m = jnp.max(logits, axis=1, keepdims=True)
    lse = m + jnp.log(jnp.sum(jnp.exp(logits - m), axis=1, keepdims=True))
    out_ref[...] = logits - lse


def _final_call(p2, self2, degp, b2r, Wl, blr):
    return pl.pallas_call(
        _final_body,
        grid=(-(-N // TB),),
        in_specs=[
            pl.BlockSpec((4, TB, 16), lambda i: (0, i, 0)),
            pl.BlockSpec((TB, 64), lambda i: (i, 0)),
            pl.BlockSpec((2, NP), lambda i: (0, 0)),
            pl.BlockSpec((1, 64), lambda i: (0, 0)),
            pl.BlockSpec((64, 64), lambda i: (0, 0)),
            pl.BlockSpec((1, 64), lambda i: (0, 0)),
        ],
        out_specs=pl.BlockSpec((TB, 64), lambda i: (i, 0)),
        out_shape=jax.ShapeDtypeStruct((N, 64), jnp.float32),
    )(p2, self2, degp, b2r, Wl, blr)


def kernel(x, edge_index, edge_weight, W1, b1, W2, b2, Wl, bl):
    src = edge_index[0]
    dst = edge_index[1]
    pad = EP - E
    pad_idx = (jnp.arange(pad, dtype=jnp.int32) % N)
    srcr = jnp.concatenate([src, pad_idx]).reshape(ER, 128)
    dstr = jnp.concatenate([dst, pad_idx]).reshape(ER, 128)
    wp = jnp.concatenate([edge_weight, jnp.zeros((pad,), jnp.float32)])
    wr2 = wp.reshape(ER, 128)

    xp = jnp.concatenate(
        [x, jnp.zeros((NP - N, 23), jnp.float32)], axis=0)
    degp = _deg_call(dstr, wr2)
    featS = _prep_call(degp, xp)
    p1 = _prop_call(2, featS, srcr, dstr, wp)
    W1p = jnp.concatenate([W1, jnp.zeros((9, 128), jnp.float32)], axis=0)
    tS, self2 = _mid_call(p1, xp, degp, W1p, b1.reshape(1, 128), W2)
    p2 = _prop_call(4, tS, srcr, dstr, wp)
    return _final_call(p2, self2, degp,
                       b2.reshape(1, 64), Wl, bl.reshape(1, 64))


# speedup vs baseline: 34.5475x; 1.0311x over previous
import functools

import jax
import jax.numpy as jnp
from jax import lax
from jax.experimental import pallas as pl
from jax.experimental.pallas import tpu as pltpu
from jax.experimental.pallas import tpu_sc as plsc

N = 100000
NP = 102400
E = 3200000
EP = 16 * 1568 * 128
ER = EP // 128
TILE_ROWS = 1568
PWR = 4
PB = PWR * 128
PNWIN = TILE_ROWS // PWR
WIN_ROWS = 16

DEG_TILE_ROWS = ER // 32
DEG_NWIN = DEG_TILE_ROWS // WIN_ROWS

TB = 2048

_mesh = plsc.VectorSubcoreMesh(core_axis_name="c", subcore_axis_name="s")


DWR = 8
DEG_TILE_ROWS2 = ER // 32
DEG_NW = DEG_TILE_ROWS2 // DWR


def _deg_body(dstr_hbm, wr2_hbm, degp_hbm, acc, dst_v, w_v, zbuf,
              sem_i, sem_s):
    c = lax.axis_index("c")
    s = lax.axis_index("s")
    wid = c * 16 + s

    @pl.when(s == 0)
    def _():
        @pl.loop(0, 6400 // 16)
        def _(i):
            zbuf[pl.ds(i * 16, 16)] = jnp.zeros((16,), jnp.float32)

        @pl.loop(0, 16)
        def _(b):
            pltpu.sync_copy(zbuf, acc.at[pl.ds(b * 6400, 6400)])

    plsc.subcore_barrier()

    row0 = wid * DEG_TILE_ROWS2

    def load(pp, i, fire):
        rw = row0 + i * DWR
        a = pltpu.make_async_copy(
            dstr_hbm.at[pl.ds(rw, DWR)], dst_v.at[pp], sem_i)
        b = pltpu.make_async_copy(
            wr2_hbm.at[pl.ds(rw, DWR)], w_v.at[pp], sem_i)
        if fire:
            a.start(); b.start()
        else:
            a.wait(); b.wait()

    def scatter(pp, fire):
        for r in range(DWR):
            if fire:
                pltpu.async_copy(
                    w_v.at[pp].at[r], acc.at[dst_v.at[pp].at[r]],
                    sem_s, add=True)
            else:
                pltpu.make_async_copy(
                    w_v.at[pp].at[r], acc.at[dst_v.at[pp].at[r]],
                    sem_s).wait()

    load(0, 0, True)

    @pl.loop(0, DEG_NW // 2)
    def _(t):
        i0 = 2 * t
        load(0, i0, False)

        @pl.when(t > 0)
        def _():
            scatter(1, False)

        load(1, i0 + 1, True)
        scatter(0, True)
        load(1, i0 + 1, False)
        scatter(0, False)

        @pl.when(t < DEG_NW // 2 - 1)
        def _():
            load(0, i0 + 2, True)

        scatter(1, True)

    scatter(1, False)

    plsc.subcore_barrier()

    @pl.when(s == 0)
    def _():
        pltpu.sync_copy(acc, degp_hbm.at[c])


def _deg_call(dstr, wr2):
    return pl.kernel(
        _deg_body,
        out_type=jax.ShapeDtypeStruct((2, NP), jnp.float32),
        mesh=_mesh,
        compiler_params=pltpu.CompilerParams(use_tc_tiling_on_sc=False),
        scratch_types=[
            pltpu.VMEM_SHARED((NP,), jnp.float32),
            pltpu.VMEM((2, DWR, 128), jnp.int32),
            pltpu.VMEM((2, DWR, 128), jnp.float32),
            pltpu.VMEM((6400,), jnp.float32),
            pltpu.SemaphoreType.DMA,
            pltpu.SemaphoreType.DMA,
        ],
    )(dstr, wr2)


def _prop_body(K, feat_hbm, srcr_hbm, dstr_hbm, wp_hbm, out_hbm,
               acc, src_v, dst_v, w_v, rows_v, zbuf,
               sem_i, sem_d, sem_g, sem_s):
    c = lax.axis_index("c")
    s = lax.axis_index("s")
    row0 = s * TILE_ROWS
    zrow0 = s * (NP // 16)

    @pl.loop(0, 128)
    def _(i):
        zbuf[i, :] = jnp.zeros((16,), jnp.float32)

    def load_srcw(pp, i, fire):
        rw = row0 + i * PWR
        a = pltpu.make_async_copy(
            srcr_hbm.at[pl.ds(rw, PWR)], src_v.at[pp], sem_i)
        b = pltpu.make_async_copy(
            wp_hbm.at[pl.ds(rw * 128, PB)], w_v.at[pp], sem_i)
        if fire:
            a.start(); b.start()
        else:
            a.wait(); b.wait()

    def load_dst(pp, i, fire):
        rw = row0 + i * PWR
        a = pltpu.make_async_copy(
            dstr_hbm.at[pl.ds(rw, PWR)], dst_v.at[pp], sem_d)
        a.start() if fire else a.wait()

    def gather(pp, kb, fire):
        for r in range(PWR):
            a = pltpu.make_async_copy(
                feat_hbm.at[kb].at[src_v.at[pp].at[r]],
                rows_v.at[pp].at[pl.ds(r * 128, 128)], sem_g)
            a.start() if fire else a.wait()

    def scatter(pp, fire):
        for r in range(PWR):
            if fire:
                pltpu.async_copy(
                    rows_v.at[pp].at[pl.ds(r * 128, 128)],
                    acc.at[dst_v.at[pp].at[r]], sem_s, add=True)
            else:
                pltpu.make_async_copy(
                    rows_v.at[pp].at[pl.ds(r * 128, 128)],
                    acc.at[dst_v.at[pp].at[r]], sem_s).wait()

    def scale(pp):
        @pl.loop(0, PB, step=16)
        def _(q):
            wv = w_v[pp, pl.ds(q, 16)]
            for u in range(16):
                rows_v[pp, q + u, :] = rows_v[pp, q + u, :] * wv[u]

    NW = PNWIN
    for j in range(K // 2):
        kb = c + 2 * j

        @pl.loop(0, 50)
        def _(b):
            pltpu.sync_copy(zbuf, acc.at[pl.ds(zrow0 + b * 128, 128)])

        plsc.subcore_barrier()

        load_srcw(0, 0, True)
        load_srcw(0, 0, False)
        gather(0, kb, True)
        load_srcw(1, 1, True)
        load_dst(0, 0, True)

        @pl.loop(0, NW // 2)
        def _(t):
            i0 = 2 * t
            gather(0, kb, False)
            load_srcw(1, i0 + 1, False)

            @pl.when(t > 0)
            def _():
                scatter(1, False)

            load_dst(1, i0 + 1, True)
            gather(1, kb, True)
            scale(0)
            load_dst(0, i0, False)
            scatter(0, True)

            @pl.when(t < NW // 2 - 1)
            def _():
                load_srcw(0, i0 + 2, True)

            gather(1, kb, False)

            @pl.when(t < NW // 2 - 1)
            def _():
                load_srcw(0, i0 + 2, False)
                scatter(0, False)
                load_dst(0, i0 + 2, True)
                gather(0, kb, True)

            @pl.when(t == NW // 2 - 1)
            def _():
                scatter(0, False)

            scale(1)
            load_dst(1, i0 + 1, False)
            scatter(1, True)

            @pl.when(t < NW // 2 - 1)
            def _():
                load_srcw(1, i0 + 3, True)

        scatter(1, False)

        plsc.subcore_barrier()

        @pl.when(s == 0)
        def _():
            pltpu.sync_copy(acc, out_hbm.at[kb])

        plsc.subcore_barrier()


def _prop_call(K, feat, srcr, dstr, wp):
    return pl.kernel(
        functools.partial(_prop_body, K),
        out_type=jax.ShapeDtypeStruct((K, NP, 16), jnp.float32),
        mesh=_mesh,
        compiler_params=pltpu.CompilerParams(use_tc_tiling_on_sc=False),
        scratch_types=[
            pltpu.VMEM_SHARED((NP, 16), jnp.float32),
            pltpu.VMEM((2, PWR, 128), jnp.int32),
            pltpu.VMEM((2, PWR, 128), jnp.int32),
            pltpu.VMEM((2, PB), jnp.float32),
            pltpu.VMEM((2, PB, 16), jnp.float32),
            pltpu.VMEM((128, 16), jnp.float32),
            pltpu.SemaphoreType.DMA,
            pltpu.SemaphoreType.DMA,
            pltpu.SemaphoreType.DMA,
            pltpu.SemaphoreType.DMA,
        ],
    )(feat, srcr, dstr, wp)


def _deg_slice(degp_ref):
    i = pl.program_id(0)
    sl = pl.ds(i * TB, TB)
    return degp_ref[0, sl] + degp_ref[1, sl] + 1.0


def _prep_body(degp_ref, x_ref, featS_ref):
    deg = _deg_slice(degp_ref)
    dv = lax.rsqrt(deg)
    xs = x_ref[...] * dv[:, None]
    featS_ref[0, :, :] = xs[:, :16]
    featS_ref[1, :, 0:7] = xs[:, 16:23]
    featS_ref[1, :, 7:16] = jnp.zeros((TB, 9), jnp.float32)


def _prep_call(degp, x):
    return pl.pallas_call(
        _prep_body,
        grid=(NP // TB,),
        in_specs=[
            pl.BlockSpec((2, NP), lambda i: (0, 0)),
            pl.BlockSpec((TB, 23), lambda i: (i, 0)),
        ],
        out_specs=pl.BlockSpec((2, TB, 16), lambda i: (0, i, 0)),
        out_shape=jax.ShapeDtypeStruct((2, NP, 16), jnp.float32),
    )(degp, x)


def _mid_body(p1_ref, x_ref, degp_ref, W1p_ref, b1_ref, W2_ref,
              tS_ref, self2_ref):
    deg = _deg_slice(degp_ref)
    dv = lax.rsqrt(deg)
    dv2 = 1.0 / deg
    p1 = p1_ref[...]
    W1p = W1p_ref[...]
    agg = (jnp.dot(p1[0], W1p[0:16], preferred_element_type=jnp.float32)
           + jnp.dot(p1[1], W1p[16:32], preferred_element_type=jnp.float32))
    selfp = jnp.dot(x_ref[...], W1p[0:23],
                    preferred_element_type=jnp.float32)
    h1 = jnp.maximum(
        agg * dv[:, None] + selfp * dv2[:, None] + b1_ref[...], 0.0)
    t = jnp.dot(h1, W2_ref[...], preferred_element_type=jnp.float32)
    ts = t * dv[:, None]
    for k in range(4):
        tS_ref[k, :, :] = ts[:, 16 * k:16 * (k + 1)]
    self2_ref[...] = t * dv2[:, None]


def _mid_call(p1, x, degp, W1p, b1r, W2):
    return pl.pallas_call(
        _mid_body,
        grid=(NP // TB,),
        in_specs=[
            pl.BlockSpec((2, TB, 16), lambda i: (0, i, 0)),
            pl.BlockSpec((TB, 23), lambda i: (i, 0)),
            pl.BlockSpec((2, NP), lambda i: (0, 0)),
            pl.BlockSpec((32, 128), lambda i: (0, 0)),
            pl.BlockSpec((1, 128), lambda i: (0, 0)),
            pl.BlockSpec((128, 64), lambda i: (0, 0)),
        ],
        out_specs=[
            pl.BlockSpec((4, TB, 16), lambda i: (0, i, 0)),
            pl.BlockSpec((TB, 64), lambda i: (i, 0)),
        ],
        out_shape=[
            jax.ShapeDtypeStruct((4, NP, 16), jnp.float32),
            jax.ShapeDtypeStruct((NP, 64), jnp.float32),
        ],
    )(p1, x, degp, W1p, b1r, W2)


def _final_body(p2_ref, self2_ref, degp_ref, b2_ref, Wl_ref, bl_ref, out_ref):
    deg = _deg_slice(degp_ref)
    dv = lax.rsqrt(deg)
    p2 = p2_ref[...]
    self2 = self2_ref[...]
    b2 = b2_ref[...]
    Wl = Wl_ref[...]
    logits = bl_ref[...]
    for k in range(4):
        h2k = jnp.maximum(
            p2[k] * dv[:, None] + self2[:, 16 * k:16 * (k + 1)]
            + b2[:, 16 * k:16 * (k + 1)], 0.0)
        logits = logits + jnp.dot(h2k, Wl[16 * k:16 * (k + 1)],
                                  preferred_element_type=jnp.float32)
    m = jnp.max(logits, axis=1, keepdims=True)
    lse = m + jnp.log(jnp.sum(jnp.exp(logits - m), axis=1, keepdims=True))
    out_ref[...] = logits - lse


def _final_call(p2, self2, degp, b2r, Wl, blr):
    return pl.pallas_call(
        _final_body,
        grid=(-(-N // TB),),
        in_specs=[
            pl.BlockSpec((4, TB, 16), lambda i: (0, i, 0)),
            pl.BlockSpec((TB, 64), lambda i: (i, 0)),
            pl.BlockSpec((2, NP), lambda i: (0, 0)),
            pl.BlockSpec((1, 64), lambda i: (0, 0)),
            pl.BlockSpec((64, 64), lambda i: (0, 0)),
            pl.BlockSpec((1, 64), lambda i: (0, 0)),
        ],
        out_specs=pl.BlockSpec((TB, 64), lambda i: (i, 0)),
        out_shape=jax.ShapeDtypeStruct((N, 64), jnp.float32),
    )(p2, self2, degp, b2r, Wl, blr)


def kernel(x, edge_index, edge_weight, W1, b1, W2, b2, Wl, bl):
    src = edge_index[0]
    dst = edge_index[1]
    pad = EP - E
    pad_idx = (jnp.arange(pad, dtype=jnp.int32) % N)
    srcr = jnp.concatenate([src, pad_idx]).reshape(ER, 128)
    dstr = jnp.concatenate([dst, pad_idx]).reshape(ER, 128)
    wp = jnp.concatenate([edge_weight, jnp.zeros((pad,), jnp.float32)])
    wr2 = wp.reshape(ER, 128)

    xp = jnp.concatenate(
        [x, jnp.zeros((NP - N, 23), jnp.float32)], axis=0)
    degp = _deg_call(dstr, wr2)
    featS = _prep_call(degp, xp)
    p1 = _prop_call(2, featS, srcr, dstr, wp)
    W1p = jnp.concatenate([W1, jnp.zeros((9, 128), jnp.float32)], axis=0)
    tS, self2 = _mid_call(p1, xp, degp, W1p, b1.reshape(1, 128), W2)
    p2 = _prop_call(4, tS, srcr, dstr, wp)
    return _final_call(p2, self2, degp,
                       b2.reshape(1, 64), Wl, bl.reshape(1, 64))
